# baseline (speedup 1.0000x reference)
import functools

import jax
import jax.numpy as jnp
from jax import lax
from jax.experimental import pallas as pl
from jax.experimental.pallas import tpu as pltpu
from jax.experimental.pallas import tpu_sc as plsc

N = 10000
NP = 10240
E = 320000
F = 128
G = 8

NC = 2
NS = 16
CHT = 80
ET = CHT * 128
EPAD = NC * NS * ET
FC0 = 96
FC1 = 2 * CHT - FC0
EB = EPAD // 128
RPT = NP // NS

_MESH = plsc.VectorSubcoreMesh(core_axis_name="c", subcore_axis_name="s")

_GDN = lax.GatherDimensionNumbers(
    offset_dims=(), collapsed_slice_dims=(0,), start_index_map=(0,))


def _bcast_lane(vec16, i):
    return lax.gather(vec16, jnp.full((16, 1), i, jnp.int32), _GDN, (1,),
                      mode=lax.GatherScatterMode.PROMISE_IN_BOUNDS)


@functools.partial(
    pl.kernel,
    out_type=jax.ShapeDtypeStruct((NC, NP), jnp.float32),
    mesh=_MESH,
    scratch_types=[
        pltpu.VMEM((2, 128), jnp.int32),
        pltpu.VMEM((128,), jnp.float32),
        pltpu.VMEM_SHARED((NP,), jnp.float32),
    ],
)
def _sc_deg(e2_hbm, w_hbm, z1_hbm, out_hbm, ev, wv, acc_sh):
    cid = lax.axis_index("c")
    sid = lax.axis_index("s")
    tb = (cid * NS + sid) * CHT
    pltpu.sync_copy(z1_hbm, acc_sh.at[pl.ds(sid * RPT, RPT)])
    plsc.subcore_barrier()

    def chunk(kk, carry):
        pltpu.sync_copy(e2_hbm.at[tb + kk], ev)
        pltpu.sync_copy(w_hbm.at[tb + kk], wv)
        pltpu.sync_copy(wv, acc_sh.at[ev.at[1]], add=True)
        return carry

    lax.fori_loop(0, CHT, chunk, 0)
    plsc.subcore_barrier()
    pltpu.sync_copy(acc_sh.at[pl.ds(sid * RPT, RPT)],
                    out_hbm.at[cid, pl.ds(sid * RPT, RPT)])


@functools.partial(
    pl.kernel,
    out_type=jax.ShapeDtypeStruct((NC, NP, F), jnp.float32),
    mesh=_MESH,
    scratch_types=[
        pltpu.VMEM((2, 128), jnp.int32),
        pltpu.VMEM((2, 128), jnp.int32),
        pltpu.VMEM((2, 128), jnp.int32),
        pltpu.VMEM((2, 128), jnp.int32),
        pltpu.VMEM((128,), jnp.float32),
        pltpu.VMEM((128,), jnp.float32),
        pltpu.VMEM((128,), jnp.float32),
        pltpu.VMEM((128,), jnp.float32),
        pltpu.VMEM((128, F), jnp.float32),
        pltpu.VMEM((128, F), jnp.float32),
        pltpu.VMEM_SHARED((NP, F), jnp.float32),
        pltpu.SemaphoreType.DMA,
        pltpu.SemaphoreType.DMA,
        pltpu.SemaphoreType.DMA,
    ],
)
def _sc_feat(y_hbm, e2_hbm, w_hbm, z2_hbm, out_hbm,
             ev0, ev1, ev2, ev3, wv0, wv1, wv2, wv3, rows_a, rows_b,
             acc_sh, sem_i, sem_g0, sem_g1):
    cid = lax.axis_index("c")
    sid = lax.axis_index("s")
    evs = [ev0, ev1, ev2, ev3]
    wvs = [wv0, wv1, wv2, wv3]
    rows = [rows_a, rows_b]
    gsems = [sem_g0, sem_g1]
    pltpu.sync_copy(z2_hbm, acc_sh.at[pl.ds(sid * RPT, RPT), :])
    plsc.subcore_barrier()

    def run(tb, cht):
        def idxload(blk, s):
            pltpu.async_copy(e2_hbm.at[tb + blk], evs[s], sem_i)
            pltpu.async_copy(w_hbm.at[tb + blk], wvs[s], sem_i)

        def idxwait(blk, s):
            pltpu.make_async_copy(e2_hbm.at[tb + blk], evs[s], sem_i).wait()
            pltpu.make_async_copy(w_hbm.at[tb + blk], wvs[s], sem_i).wait()

        def gat_start(s, r):
            pltpu.async_copy(y_hbm.at[evs[s].at[0]], rows[r], gsems[r])

        def gat_wait(s, r):
            pltpu.make_async_copy(y_hbm.at[evs[s].at[0]], rows[r],
                                  gsems[r]).wait()

        def scale_scatter(s, r):
            def scale(g, c2):
                wchunk = wvs[s][pl.ds(g * 16, 16)]
                for i in range(16):
                    wb = _bcast_lane(wchunk, i)
                    rr = g * 16 + i
                    for j in range(8):
                        rows[r][rr, pl.ds(j * 16, 16)] = \
                            rows[r][rr, pl.ds(j * 16, 16)] * wb
                return c2

            lax.fori_loop(0, 8, scale, 0)
            pltpu.sync_copy(rows[r], acc_sh.at[evs[s].at[1]], add=True)

        idxload(0, 0)
        idxwait(0, 0)
        gat_start(0, 0)
        idxload(1, 1)

        def body(it, carry):
            for b in range(4):
                kk = it * 4 + b
                s = b
                r = b % 2
                ns = (b + 1) % 4
                nr = (b + 1) % 2

                @pl.when(kk + 1 < cht)
                def _():
                    idxwait(kk + 1, ns)
                    gat_start(ns, nr)

                gat_wait(s, r)
                scale_scatter(s, r)

                @pl.when(kk + 2 < cht)
                def _():
                    idxload(kk + 2, (b + 2) % 4)
            return carry

        lax.fori_loop(0, cht // 4, body, 0)

    @pl.when(cid == 0)
    def _():
        run(sid * FC0, FC0)

    @pl.when(cid == 1)
    def _():
        run(NS * FC0 + sid * FC1, FC1)

    plsc.subcore_barrier()
    pltpu.sync_copy(acc_sh.at[pl.ds(sid * RPT, RPT), :],
                    out_hbm.at[cid, pl.ds(sid * RPT, RPT), :])


@functools.partial(
    pl.kernel,
    out_type=jax.ShapeDtypeStruct((NC, NP), jnp.float32),
    mesh=_MESH,
    scratch_types=[
        pltpu.VMEM((2, 128), jnp.int32),
        pltpu.VMEM((2, 128), jnp.int32),
        pltpu.VMEM((128,), jnp.float32),
        pltpu.VMEM((128,), jnp.float32),
        pltpu.VMEM((128,), jnp.float32),
        pltpu.VMEM((128,), jnp.float32),
        pltpu.VMEM((128,), jnp.float32),
        pltpu.VMEM((128,), jnp.float32),
        pltpu.VMEM_SHARED((NP,), jnp.float32),
        pltpu.SemaphoreType.DMA,
        pltpu.SemaphoreType.DMA,
        pltpu.SemaphoreType.DMA,
    ],
)
def _sc_smsg(z_hbm, e2_hbm, w_hbm, z1_hbm, out_hbm,
             eva, evb, wva, wvb, za, zb, msga, msgb,
             acc_sh, sem_i, sem_ga, sem_gb, ):
    cid = lax.axis_index("c")
    sid = lax.axis_index("s")
    tb = (cid * NS + sid) * CHT
    evs = [eva, evb]
    wvs = [wva, wvb]
    zs = [za, zb]
    msgs = [msga, msgb]
    gsems = [sem_ga, sem_gb]
    pltpu.sync_copy(z1_hbm, acc_sh.at[pl.ds(sid * RPT, RPT)])
    plsc.subcore_barrier()

    def idxload(blk, s):
        pltpu.async_copy(e2_hbm.at[tb + blk], evs[s], sem_i)
        pltpu.async_copy(w_hbm.at[tb + blk], wvs[s], sem_i)

    def idxwait(blk, s):
        pltpu.make_async_copy(e2_hbm.at[tb + blk], evs[s], sem_i).wait()
        pltpu.make_async_copy(w_hbm.at[tb + blk], wvs[s], sem_i).wait()

    idxload(0, 0)
    idxwait(0, 0)
    pltpu.async_copy(z_hbm.at[evs[0].at[0]], zs[0], gsems[0])
    idxload(1, 1)

    def body(it, carry):
        for b in range(2):
            kk = it * 2 + b
            s = b
            ns = (b + 1) % 2

            @pl.when(kk + 1 < CHT)
            def _():
                idxwait(kk + 1, ns)
                pltpu.async_copy(z_hbm.at[evs[ns].at[0]], zs[ns], gsems[ns])

            pltpu.make_async_copy(z_hbm.at[evs[s].at[0]], zs[s],
                                  gsems[s]).wait()
            for j in range(8):
                msgs[s][pl.ds(j * 16, 16)] = \
                    zs[s][pl.ds(j * 16, 16)] * wvs[s][pl.ds(j * 16, 16)]
            pltpu.sync_copy(msgs[s], acc_sh.at[evs[s].at[1]], add=True)

            @pl.when(kk + 2 < CHT)
            def _():
                idxload(kk + 2, s)
        return carry

    lax.fori_loop(0, CHT // 2, body, 0)
    plsc.subcore_barrier()
    pltpu.sync_copy(acc_sh.at[pl.ds(sid * RPT, RPT)],
                    out_hbm.at[cid, pl.ds(sid * RPT, RPT)])


@functools.partial(
    pl.kernel,
    out_type=(jax.ShapeDtypeStruct((EB, 128), jnp.float32),
              jax.ShapeDtypeStruct((NC, NP), jnp.float32)),
    mesh=_MESH,
    scratch_types=[
        pltpu.VMEM((2, 128), jnp.int32),
        pltpu.VMEM((2, 128), jnp.int32),
        pltpu.VMEM((128,), jnp.float32),
        pltpu.VMEM((128,), jnp.float32),
        pltpu.VMEM((128,), jnp.float32),
        pltpu.VMEM((128,), jnp.float32),
        pltpu.VMEM((128,), jnp.float32),
        pltpu.VMEM((128,), jnp.float32),
        pltpu.VMEM((128,), jnp.float32),
        pltpu.VMEM((128,), jnp.float32),
        pltpu.VMEM_SHARED((NP,), jnp.float32),
        pltpu.SemaphoreType.DMA,
        pltpu.SemaphoreType.DMA,
        pltpu.SemaphoreType.DMA,
    ],
)
def _sc_w2deg(sel_hbm, e2_hbm, w_hbm, z1_hbm, w2_hbm, out_hbm,
              eva, evb, wva, wvb, sra, srb, sca, scb, w2a, w2b,
              acc_sh, sem_i, sem_ga, sem_gb):
    cid = lax.axis_index("c")
    sid = lax.axis_index("s")
    tb = (cid * NS + sid) * CHT
    evs = [eva, evb]
    wvs = [wva, wvb]
    srs = [sra, srb]
    scs = [sca, scb]
    w2s = [w2a, w2b]
    gsems = [sem_ga, sem_gb]
    pltpu.sync_copy(z1_hbm, acc_sh.at[pl.ds(sid * RPT, RPT)])
    plsc.subcore_barrier()

    def idxload(blk, s):
        pltpu.async_copy(e2_hbm.at[tb + blk], evs[s], sem_i)
        pltpu.async_copy(w_hbm.at[tb + blk], wvs[s], sem_i)

    def idxwait(blk, s):
        pltpu.make_async_copy(e2_hbm.at[tb + blk], evs[s], sem_i).wait()
        pltpu.make_async_copy(w_hbm.at[tb + blk], wvs[s], sem_i).wait()

    def gat_start(s):
        pltpu.async_copy(sel_hbm.at[evs[s].at[0]], srs[s], gsems[s])
        pltpu.async_copy(sel_hbm.at[evs[s].at[1]], scs[s], gsems[s])

    def gat_wait(s):
        pltpu.make_async_copy(sel_hbm.at[evs[s].at[0]], srs[s],
                              gsems[s]).wait()
        pltpu.make_async_copy(sel_hbm.at[evs[s].at[1]], scs[s],
                              gsems[s]).wait()

    idxload(0, 0)
    idxwait(0, 0)
    gat_start(0)
    idxload(1, 1)

    def body(it, carry):
        for b in range(2):
            kk = it * 2 + b
            s = b
            ns = (b + 1) % 2

            @pl.when(kk + 1 < CHT)
            def _():
                idxwait(kk + 1, ns)
                gat_start(ns)

            gat_wait(s)
            for j in range(8):
                w2s[s][pl.ds(j * 16, 16)] = wvs[s][pl.ds(j * 16, 16)] * \
                    srs[s][pl.ds(j * 16, 16)] * scs[s][pl.ds(j * 16, 16)]
            pltpu.sync_copy(w2s[s], w2_hbm.at[tb + kk])
            pltpu.sync_copy(w2s[s], acc_sh.at[evs[s].at[1]], add=True)

            @pl.when(kk + 2 < CHT)
            def _():
                idxload(kk + 2, s)
        return carry

    lax.fori_loop(0, CHT // 2, body, 0)
    plsc.subcore_barrier()
    pltpu.sync_copy(acc_sh.at[pl.ds(sid * RPT, RPT)],
                    out_hbm.at[cid, pl.ds(sid * RPT, RPT)])


_BM = 1024
_GRID = NP // _BM


def _mm_body(x_ref, w_ref, xw_ref):
    xw_ref[...] = jnp.dot(x_ref[...], w_ref[...],
                          preferred_element_type=jnp.float32)


def _tc_mm(x, W):
    return pl.pallas_call(
        _mm_body,
        grid=(_GRID,),
        in_specs=[
            pl.BlockSpec((_BM, F), lambda i: (i, 0)),
            pl.BlockSpec((F, F), lambda i: (0, 0)),
        ],
        out_specs=pl.BlockSpec((_BM, F), lambda i: (i, 0)),
        out_shape=jax.ShapeDtypeStruct((NP, F), jnp.float32),
    )(x, W)


def _post_body(a0_ref, a1_ref, xw_ref, d0_ref, d1_ref, b_ref, wp_ref,
               h_ref, sp_ref, z_ref):
    dinv = lax.rsqrt(d0_ref[...] + d1_ref[...] + 1.0)
    xw = xw_ref[...]
    h = jnp.maximum(
        dinv * (a0_ref[...] + a1_ref[...]) + dinv * dinv * xw + b_ref[...],
        0.0)
    sp = jnp.sum(h * wp_ref[...], axis=1, keepdims=True)
    h_ref[...] = h
    sp_ref[...] = sp
    z_ref[...] = dinv[:, 0:1] * sp


def _tc_post(a0, a1, xw, d0, d1, b, wp):
    return pl.pallas_call(
        _post_body,
        grid=(_GRID,),
        in_specs=[
            pl.BlockSpec((_BM, F), lambda i: (i, 0)),
            pl.BlockSpec((_BM, F), lambda i: (i, 0)),
            pl.BlockSpec((_BM, F), lambda i: (i, 0)),
            pl.BlockSpec((_BM, 1), lambda i: (i, 0)),
            pl.BlockSpec((_BM, 1), lambda i: (i, 0)),
            pl.BlockSpec((1, F), lambda i: (0, 0)),
            pl.BlockSpec((1, F), lambda i: (0, 0)),
        ],
        out_specs=[
            pl.BlockSpec((_BM, F), lambda i: (i, 0)),
            pl.BlockSpec((_BM, 1), lambda i: (i, 0)),
            pl.BlockSpec((_BM, 1), lambda i: (i, 0)),
        ],
        out_shape=[
            jax.ShapeDtypeStruct((NP, F), jnp.float32),
            jax.ShapeDtypeStruct((NP, 1), jnp.float32),
            jax.ShapeDtypeStruct((NP, 1), jnp.float32),
        ],
    )(a0, a1, xw, d0, d1, b, wp)


def _topk_body(s0_ref, s1_ref, d0_ref, d1_ref, sp_ref, bp_ref, valid_ref,
               batch_ref, score_ref, sel_ref):
    dinv = lax.rsqrt(d0_ref[...] + d1_ref[...] + 1.0)
    score = dinv * (s0_ref[...] + s1_ref[...]) + dinv * dinv * sp_ref[...] \
        + bp_ref[...]
    score_ref[...] = score

    bits = lax.bitcast_convert_type(score, jnp.uint32)
    key = jnp.where(bits >> 31 != 0, ~bits, bits | jnp.uint32(0x80000000))

    gidx = lax.broadcasted_iota(jnp.int32, (G, 1), 0)
    gm = (batch_ref[...] == gidx) & (valid_ref[...] > 0)
    counts = jnp.sum(gm.astype(jnp.int32), axis=1, keepdims=True)
    k = (counts + 1) // 2

    def bit_step(t, prefix):
        b = (31 - t).astype(jnp.uint32)
        cand = prefix | lax.shift_left(jnp.uint32(1), b)
        cnt = jnp.sum((gm & (key >= cand)).astype(jnp.int32), axis=1,
                      keepdims=True)
        return jnp.where(cnt >= k, cand, prefix)

    T = lax.fori_loop(0, 32, bit_step, jnp.zeros((G, 1), jnp.uint32))
    cnt_gt = jnp.sum((gm & (key > T)).astype(jnp.int32), axis=1, keepdims=True)
    m = k - cnt_gt
    tie = gm & (key == T)
    idxv = lax.broadcasted_iota(jnp.int32, (1, NP), 1)

    def idx_step(t, ipref):
        cand = ipref | lax.shift_left(1, 13 - t)
        cnt_less = jnp.sum((tie & (idxv < cand)).astype(jnp.int32), axis=1,
                           keepdims=True)
        return jnp.where(cnt_less < m, cand, ipref)

    istar = lax.fori_loop(0, 14, idx_step, jnp.zeros((G, 1), jnp.int32))
    sel_g = (gm & (key > T)) | (tie & (idxv <= istar) & (m > 0))
    sel_ref[...] = jnp.any(sel_g, axis=0, keepdims=True).astype(jnp.float32)


def _tc_topk(s0, s1, d0, d1, sp, bp, valid, batch):
    return pl.pallas_call(
        _topk_body,
        out_shape=[
            jax.ShapeDtypeStruct((1, NP), jnp.float32),
            jax.ShapeDtypeStruct((1, NP), jnp.float32),
        ],
    )(s0, s1, d0, d1, sp, bp, valid, batch)


def _gate_mm_body(h_ref, s_ref, w_ref, hg_ref, xw2_ref):
    hg = h_ref[...] * jnp.tanh(s_ref[...])
    hg_ref[...] = hg
    xw2_ref[...] = jnp.dot(hg, w_ref[...], preferred_element_type=jnp.float32)


def _tc_gate_mm(h, score, W):
    return pl.pallas_call(
        _gate_mm_body,
        grid=(_GRID,),
        in_specs=[
            pl.BlockSpec((_BM, F), lambda i: (i, 0)),
            pl.BlockSpec((_BM, 1), lambda i: (i, 0)),
            pl.BlockSpec((F, F), lambda i: (0, 0)),
        ],
        out_specs=[
            pl.BlockSpec((_BM, F), lambda i: (i, 0)),
            pl.BlockSpec((_BM, F), lambda i: (i, 0)),
        ],
        out_shape=[
            jax.ShapeDtypeStruct((NP, F), jnp.float32),
            jax.ShapeDtypeStruct((NP, F), jnp.float32),
        ],
    )(h, score, W)


def _scale_body(xw_ref, d0_ref, d1_ref, y_ref):
    dinv = lax.rsqrt(d0_ref[...] + d1_ref[...] + 1.0)
    y_ref[...] = dinv * xw_ref[...]


def _tc_scale(xw, d0, d1):
    return pl.pallas_call(
        _scale_body,
        grid=(_GRID,),
        in_specs=[
            pl.BlockSpec((_BM, F), lambda i: (i, 0)),
            pl.BlockSpec((_BM, 1), lambda i: (i, 0)),
            pl.BlockSpec((_BM, 1), lambda i: (i, 0)),
        ],
        out_specs=pl.BlockSpec((_BM, F), lambda i: (i, 0)),
        out_shape=jax.ShapeDtypeStruct((NP, F), jnp.float32),
    )(xw, d0, d1)


def _gate_body(h_ref, s_ref, hg_ref):
    hg_ref[...] = h_ref[...] * jnp.tanh(s_ref[...])


def _tc_gate(h, score):
    return pl.pallas_call(
        _gate_body,
        grid=(_GRID,),
        in_specs=[
            pl.BlockSpec((_BM, F), lambda i: (i, 0)),
            pl.BlockSpec((_BM, 1), lambda i: (i, 0)),
        ],
        out_specs=pl.BlockSpec((_BM, F), lambda i: (i, 0)),
        out_shape=jax.ShapeDtypeStruct((NP, F), jnp.float32),
    )(h, score)


def _gmp_body_first(hg_ref, sel_ref, batch_ref, out_ref):
    _gmp_common(hg_ref, sel_ref, batch_ref, out_ref, None)


def _gmp_body_final(hg_ref, sel_ref, batch_ref, xprev_ref, out_ref):
    _gmp_common(hg_ref, sel_ref, batch_ref, out_ref, xprev_ref)


def _gmp_common(hg_ref, sel_ref, batch_ref, out_ref, xprev_ref):
    hg = hg_ref[...]
    selm = sel_ref[...] > 0
    rows = []
    for g in range(G):
        mask = (batch_ref[...] == g) & selm
        mx = jnp.max(jnp.where(mask, hg, -jnp.inf), axis=0, keepdims=True)
        sm = jnp.sum(jnp.where(mask, hg, 0.0), axis=0, keepdims=True)
        cnt = jnp.maximum(jnp.sum(mask.astype(jnp.float32)), 1.0)
        rows.append(jnp.concatenate([mx, sm / cnt], axis=1))
    res = jnp.concatenate(rows, axis=0)
    if xprev_ref is not None:
        res = (xprev_ref[...] + res) * 0.5
    out_ref[...] = res


def _tc_gmp_first(hg, sel, batch):
    return pl.pallas_call(
        _gmp_body_first,
        out_shape=jax.ShapeDtypeStruct((G, 2 * F), jnp.float32),
    )(hg, sel, batch)


def _tc_gmp_final(hg, sel, batch, xprev):
    return pl.pallas_call(
        _gmp_body_final,
        out_shape=jax.ShapeDtypeStruct((G, 2 * F), jnp.float32),
    )(hg, sel, batch, xprev)


def kernel(x, edge_index, edge_attr, batch, W1, b1, Wp1, bp1, W2, b2, Wp2, bp2):
    f32 = jnp.float32
    row = edge_index[0]
    col = edge_index[1]
    epad = EPAD - E
    rowp = jnp.concatenate([row, jnp.zeros((epad,), row.dtype)])
    colp = jnp.concatenate([col, jnp.zeros((epad,), col.dtype)])
    e2 = jnp.concatenate([rowp.reshape(EB, 1, 128), colp.reshape(EB, 1, 128)],
                         axis=1)
    wp_ = jnp.concatenate([edge_attr, jnp.zeros((epad,), f32)]).reshape(EB, 128)
    xp = jnp.concatenate([x, jnp.zeros((NP - N, F), f32)], axis=0)
    batchp = jnp.concatenate([batch, jnp.full((NP - N,), G, batch.dtype)])
    batch_r = batchp.reshape(1, NP)
    batch_c = batchp.reshape(NP, 1)
    z1d = jnp.zeros((RPT,), f32)
    z2d = jnp.zeros((RPT, F), f32)
    ones_r = jnp.ones((1, NP), f32)
    b1_2 = b1.reshape(1, F)
    b2_2 = b2.reshape(1, F)
    wp1_2 = Wp1.reshape(1, F)
    wp2_2 = Wp2.reshape(1, F)
    bp1_2 = bp1.reshape(1, 1)
    bp2_2 = bp2.reshape(1, 1)

    xw1 = _tc_mm(xp, W1)
    deg_p = _sc_deg(e2, wp_, z1d)
    d0 = deg_p[0].reshape(NP, 1)
    d1 = deg_p[1].reshape(NP, 1)
    y1 = _tc_scale(xw1, d0, d1)
    acc_p = _sc_feat(y1, e2, wp_, z2d)
    h, sp1, z1v = _tc_post(acc_p[0], acc_p[1], xw1, d0, d1, b1_2, wp1_2)
    sacc_p = _sc_smsg(z1v.reshape(NP), e2, wp_, z1d)
    score1, sel1 = _tc_topk(sacc_p[0].reshape(1, NP), sacc_p[1].reshape(1, NP),
                            d0.reshape(1, NP), d1.reshape(1, NP),
                            sp1.reshape(1, NP), bp1_2, ones_r, batch_r)
    hg, xw2 = _tc_gate_mm(h, score1.reshape(NP, 1), W2)
    x1 = _tc_gmp_first(hg, sel1.reshape(NP, 1), batch_c)

    w2, deg2_p = _sc_w2deg(sel1.reshape(NP), e2, wp_, z1d)
    e0 = deg2_p[0].reshape(NP, 1)
    e1 = deg2_p[1].reshape(NP, 1)
    y2 = _tc_scale(xw2, e0, e1)
    acc2_p = _sc_feat(y2, e2, w2, z2d)
    h2, sp2, z2v = _tc_post(acc2_p[0], acc2_p[1], xw2, e0, e1, b2_2, wp2_2)
    sacc2_p = _sc_smsg(z2v.reshape(NP), e2, w2, z1d)
    score2, sel2 = _tc_topk(sacc2_p[0].reshape(1, NP),
                            sacc2_p[1].reshape(1, NP),
                            e0.reshape(1, NP), e1.reshape(1, NP),
                            sp2.reshape(1, NP), bp2_2, sel1, batch_r)
    h3g = _tc_gate(h2, score2.reshape(NP, 1))
    return _tc_gmp_final(h3g, sel2.reshape(NP, 1), batch_c, x1)


# speedup vs baseline: 1.0171x; 1.0171x over previous
import functools

import jax
import jax.numpy as jnp
from jax import lax
from jax.experimental import pallas as pl
from jax.experimental.pallas import tpu as pltpu
from jax.experimental.pallas import tpu_sc as plsc

N = 10000
NP = 10240
E = 320000
F = 128
G = 8

NC = 2
NS = 16
CHT = 80
ET = CHT * 128
EPAD = NC * NS * ET
FC0 = 128
FC1 = 2 * CHT - FC0
EB = EPAD // 128
RPT = NP // NS

_MESH = plsc.VectorSubcoreMesh(core_axis_name="c", subcore_axis_name="s")

_GDN = lax.GatherDimensionNumbers(
    offset_dims=(), collapsed_slice_dims=(0,), start_index_map=(0,))


def _bcast_lane(vec16, i):
    return lax.gather(vec16, jnp.full((16, 1), i, jnp.int32), _GDN, (1,),
                      mode=lax.GatherScatterMode.PROMISE_IN_BOUNDS)


@functools.partial(
    pl.kernel,
    out_type=jax.ShapeDtypeStruct((NC, NP), jnp.float32),
    mesh=_MESH,
    scratch_types=[
        pltpu.VMEM((2, 128), jnp.int32),
        pltpu.VMEM((128,), jnp.float32),
        pltpu.VMEM_SHARED((NP,), jnp.float32),
    ],
)
def _sc_deg(e2_hbm, w_hbm, z1_hbm, out_hbm, ev, wv, acc_sh):
    cid = lax.axis_index("c")
    sid = lax.axis_index("s")
    tb = (cid * NS + sid) * CHT
    pltpu.sync_copy(z1_hbm, acc_sh.at[pl.ds(sid * RPT, RPT)])
    plsc.subcore_barrier()

    def chunk(kk, carry):
        pltpu.sync_copy(e2_hbm.at[tb + kk], ev)
        pltpu.sync_copy(w_hbm.at[tb + kk], wv)
        pltpu.sync_copy(wv, acc_sh.at[ev.at[1]], add=True)
        return carry

    lax.fori_loop(0, CHT, chunk, 0)
    plsc.subcore_barrier()
    pltpu.sync_copy(acc_sh.at[pl.ds(sid * RPT, RPT)],
                    out_hbm.at[cid, pl.ds(sid * RPT, RPT)])


@functools.partial(
    pl.kernel,
    out_type=jax.ShapeDtypeStruct((NC, NP, F), jnp.float32),
    mesh=_MESH,
    scratch_types=[
        pltpu.VMEM((2, 128), jnp.int32),
        pltpu.VMEM((2, 128), jnp.int32),
        pltpu.VMEM((2, 128), jnp.int32),
        pltpu.VMEM((2, 128), jnp.int32),
        pltpu.VMEM((128,), jnp.float32),
        pltpu.VMEM((128,), jnp.float32),
        pltpu.VMEM((128,), jnp.float32),
        pltpu.VMEM((128,), jnp.float32),
        pltpu.VMEM((128, F), jnp.float32),
        pltpu.VMEM((128, F), jnp.float32),
        pltpu.VMEM_SHARED((NP, F), jnp.float32),
        pltpu.SemaphoreType.DMA,
        pltpu.SemaphoreType.DMA,
        pltpu.SemaphoreType.DMA,
    ],
)
def _sc_feat(y_hbm, e2_hbm, w_hbm, z2_hbm, out_hbm,
             ev0, ev1, ev2, ev3, wv0, wv1, wv2, wv3, rows_a, rows_b,
             acc_sh, sem_i, sem_g0, sem_g1):
    cid = lax.axis_index("c")
    sid = lax.axis_index("s")
    evs = [ev0, ev1, ev2, ev3]
    wvs = [wv0, wv1, wv2, wv3]
    rows = [rows_a, rows_b]
    gsems = [sem_g0, sem_g1]
    pltpu.sync_copy(z2_hbm, acc_sh.at[pl.ds(sid * RPT, RPT), :])
    plsc.subcore_barrier()

    def run(tb, cht):
        def idxload(blk, s):
            pltpu.async_copy(e2_hbm.at[tb + blk], evs[s], sem_i)
            pltpu.async_copy(w_hbm.at[tb + blk], wvs[s], sem_i)

        def idxwait(blk, s):
            pltpu.make_async_copy(e2_hbm.at[tb + blk], evs[s], sem_i).wait()
            pltpu.make_async_copy(w_hbm.at[tb + blk], wvs[s], sem_i).wait()

        def gat_start(s, r):
            pltpu.async_copy(y_hbm.at[evs[s].at[0]], rows[r], gsems[r])

        def gat_wait(s, r):
            pltpu.make_async_copy(y_hbm.at[evs[s].at[0]], rows[r],
                                  gsems[r]).wait()

        def scale_scatter(s, r):
            def scale(g, c2):
                wchunk = wvs[s][pl.ds(g * 16, 16)]
                for i in range(16):
                    wb = _bcast_lane(wchunk, i)
                    rr = g * 16 + i
                    for j in range(8):
                        rows[r][rr, pl.ds(j * 16, 16)] = \
                            rows[r][rr, pl.ds(j * 16, 16)] * wb
                return c2

            lax.fori_loop(0, 8, scale, 0)
            pltpu.sync_copy(rows[r], acc_sh.at[evs[s].at[1]], add=True)

        idxload(0, 0)
        idxwait(0, 0)
        gat_start(0, 0)
        idxload(1, 1)

        def body(it, carry):
            for b in range(4):
                kk = it * 4 + b
                s = b
                r = b % 2
                ns = (b + 1) % 4
                nr = (b + 1) % 2

                @pl.when(kk + 1 < cht)
                def _():
                    idxwait(kk + 1, ns)
                    gat_start(ns, nr)

                gat_wait(s, r)
                scale_scatter(s, r)

                @pl.when(kk + 2 < cht)
                def _():
                    idxload(kk + 2, (b + 2) % 4)
            return carry

        lax.fori_loop(0, cht // 4, body, 0)

    @pl.when(cid == 0)
    def _():
        run(sid * FC0, FC0)

    @pl.when(cid == 1)
    def _():
        run(NS * FC0 + sid * FC1, FC1)

    plsc.subcore_barrier()
    pltpu.sync_copy(acc_sh.at[pl.ds(sid * RPT, RPT), :],
                    out_hbm.at[cid, pl.ds(sid * RPT, RPT), :])


@functools.partial(
    pl.kernel,
    out_type=jax.ShapeDtypeStruct((NC, NP), jnp.float32),
    mesh=_MESH,
    scratch_types=[
        pltpu.VMEM((2, 128), jnp.int32),
        pltpu.VMEM((2, 128), jnp.int32),
        pltpu.VMEM((128,), jnp.float32),
        pltpu.VMEM((128,), jnp.float32),
        pltpu.VMEM((128,), jnp.float32),
        pltpu.VMEM((128,), jnp.float32),
        pltpu.VMEM((128,), jnp.float32),
        pltpu.VMEM((128,), jnp.float32),
        pltpu.VMEM_SHARED((NP,), jnp.float32),
        pltpu.SemaphoreType.DMA,
        pltpu.SemaphoreType.DMA,
        pltpu.SemaphoreType.DMA,
    ],
)
def _sc_smsg(z_hbm, e2_hbm, w_hbm, z1_hbm, out_hbm,
             eva, evb, wva, wvb, za, zb, msga, msgb,
             acc_sh, sem_i, sem_ga, sem_gb, ):
    cid = lax.axis_index("c")
    sid = lax.axis_index("s")
    tb = (cid * NS + sid) * CHT
    evs = [eva, evb]
    wvs = [wva, wvb]
    zs = [za, zb]
    msgs = [msga, msgb]
    gsems = [sem_ga, sem_gb]
    pltpu.sync_copy(z1_hbm, acc_sh.at[pl.ds(sid * RPT, RPT)])
    plsc.subcore_barrier()

    def idxload(blk, s):
        pltpu.async_copy(e2_hbm.at[tb + blk], evs[s], sem_i)
        pltpu.async_copy(w_hbm.at[tb + blk], wvs[s], sem_i)

    def idxwait(blk, s):
        pltpu.make_async_copy(e2_hbm.at[tb + blk], evs[s], sem_i).wait()
        pltpu.make_async_copy(w_hbm.at[tb + blk], wvs[s], sem_i).wait()

    idxload(0, 0)
    idxwait(0, 0)
    pltpu.async_copy(z_hbm.at[evs[0].at[0]], zs[0], gsems[0])
    idxload(1, 1)

    def body(it, carry):
        for b in range(2):
            kk = it * 2 + b
            s = b
            ns = (b + 1) % 2

            @pl.when(kk + 1 < CHT)
            def _():
                idxwait(kk + 1, ns)
                pltpu.async_copy(z_hbm.at[evs[ns].at[0]], zs[ns], gsems[ns])

            pltpu.make_async_copy(z_hbm.at[evs[s].at[0]], zs[s],
                                  gsems[s]).wait()
            for j in range(8):
                msgs[s][pl.ds(j * 16, 16)] = \
                    zs[s][pl.ds(j * 16, 16)] * wvs[s][pl.ds(j * 16, 16)]
            pltpu.sync_copy(msgs[s], acc_sh.at[evs[s].at[1]], add=True)

            @pl.when(kk + 2 < CHT)
            def _():
                idxload(kk + 2, s)
        return carry

    lax.fori_loop(0, CHT // 2, body, 0)
    plsc.subcore_barrier()
    pltpu.sync_copy(acc_sh.at[pl.ds(sid * RPT, RPT)],
                    out_hbm.at[cid, pl.ds(sid * RPT, RPT)])


@functools.partial(
    pl.kernel,
    out_type=(jax.ShapeDtypeStruct((EB, 128), jnp.float32),
              jax.ShapeDtypeStruct((NC, NP), jnp.float32)),
    mesh=_MESH,
    scratch_types=[
        pltpu.VMEM((2, 128), jnp.int32),
        pltpu.VMEM((2, 128), jnp.int32),
        pltpu.VMEM((128,), jnp.float32),
        pltpu.VMEM((128,), jnp.float32),
        pltpu.VMEM((128,), jnp.float32),
        pltpu.VMEM((128,), jnp.float32),
        pltpu.VMEM((128,), jnp.float32),
        pltpu.VMEM((128,), jnp.float32),
        pltpu.VMEM((128,), jnp.float32),
        pltpu.VMEM((128,), jnp.float32),
        pltpu.VMEM_SHARED((NP,), jnp.float32),
        pltpu.SemaphoreType.DMA,
        pltpu.SemaphoreType.DMA,
        pltpu.SemaphoreType.DMA,
    ],
)
def _sc_w2deg(sel_hbm, e2_hbm, w_hbm, z1_hbm, w2_hbm, out_hbm,
              eva, evb, wva, wvb, sra, srb, sca, scb, w2a, w2b,
              acc_sh, sem_i, sem_ga, sem_gb):
    cid = lax.axis_index("c")
    sid = lax.axis_index("s")
    tb = (cid * NS + sid) * CHT
    evs = [eva, evb]
    wvs = [wva, wvb]
    srs = [sra, srb]
    scs = [sca, scb]
    w2s = [w2a, w2b]
    gsems = [sem_ga, sem_gb]
    pltpu.sync_copy(z1_hbm, acc_sh.at[pl.ds(sid * RPT, RPT)])
    plsc.subcore_barrier()

    def idxload(blk, s):
        pltpu.async_copy(e2_hbm.at[tb + blk], evs[s], sem_i)
        pltpu.async_copy(w_hbm.at[tb + blk], wvs[s], sem_i)

    def idxwait(blk, s):
        pltpu.make_async_copy(e2_hbm.at[tb + blk], evs[s], sem_i).wait()
        pltpu.make_async_copy(w_hbm.at[tb + blk], wvs[s], sem_i).wait()

    def gat_start(s):
        pltpu.async_copy(sel_hbm.at[evs[s].at[0]], srs[s], gsems[s])
        pltpu.async_copy(sel_hbm.at[evs[s].at[1]], scs[s], gsems[s])

    def gat_wait(s):
        pltpu.make_async_copy(sel_hbm.at[evs[s].at[0]], srs[s],
                              gsems[s]).wait()
        pltpu.make_async_copy(sel_hbm.at[evs[s].at[1]], scs[s],
                              gsems[s]).wait()

    idxload(0, 0)
    idxwait(0, 0)
    gat_start(0)
    idxload(1, 1)

    def body(it, carry):
        for b in range(2):
            kk = it * 2 + b
            s = b
            ns = (b + 1) % 2

            @pl.when(kk + 1 < CHT)
            def _():
                idxwait(kk + 1, ns)
                gat_start(ns)

            gat_wait(s)
            for j in range(8):
                w2s[s][pl.ds(j * 16, 16)] = wvs[s][pl.ds(j * 16, 16)] * \
                    srs[s][pl.ds(j * 16, 16)] * scs[s][pl.ds(j * 16, 16)]
            pltpu.sync_copy(w2s[s], w2_hbm.at[tb + kk])
            pltpu.sync_copy(w2s[s], acc_sh.at[evs[s].at[1]], add=True)

            @pl.when(kk + 2 < CHT)
            def _():
                idxload(kk + 2, s)
        return carry

    lax.fori_loop(0, CHT // 2, body, 0)
    plsc.subcore_barrier()
    pltpu.sync_copy(acc_sh.at[pl.ds(sid * RPT, RPT)],
                    out_hbm.at[cid, pl.ds(sid * RPT, RPT)])


_BM = 1024
_GRID = NP // _BM


def _mm_body(x_ref, w_ref, xw_ref):
    xw_ref[...] = jnp.dot(x_ref[...], w_ref[...],
                          preferred_element_type=jnp.float32)


def _tc_mm(x, W):
    return pl.pallas_call(
        _mm_body,
        grid=(_GRID,),
        in_specs=[
            pl.BlockSpec((_BM, F), lambda i: (i, 0)),
            pl.BlockSpec((F, F), lambda i: (0, 0)),
        ],
        out_specs=pl.BlockSpec((_BM, F), lambda i: (i, 0)),
        out_shape=jax.ShapeDtypeStruct((NP, F), jnp.float32),
    )(x, W)


def _post_body(a0_ref, a1_ref, xw_ref, d0_ref, d1_ref, b_ref, wp_ref,
               h_ref, sp_ref, z_ref):
    dinv = lax.rsqrt(d0_ref[...] + d1_ref[...] + 1.0)
    xw = xw_ref[...]
    h = jnp.maximum(
        dinv * (a0_ref[...] + a1_ref[...]) + dinv * dinv * xw + b_ref[...],
        0.0)
    sp = jnp.sum(h * wp_ref[...], axis=1, keepdims=True)
    h_ref[...] = h
    sp_ref[...] = sp
    z_ref[...] = dinv[:, 0:1] * sp


def _tc_post(a0, a1, xw, d0, d1, b, wp):
    return pl.pallas_call(
        _post_body,
        grid=(_GRID,),
        in_specs=[
            pl.BlockSpec((_BM, F), lambda i: (i, 0)),
            pl.BlockSpec((_BM, F), lambda i: (i, 0)),
            pl.BlockSpec((_BM, F), lambda i: (i, 0)),
            pl.BlockSpec((_BM, 1), lambda i: (i, 0)),
            pl.BlockSpec((_BM, 1), lambda i: (i, 0)),
            pl.BlockSpec((1, F), lambda i: (0, 0)),
            pl.BlockSpec((1, F), lambda i: (0, 0)),
        ],
        out_specs=[
            pl.BlockSpec((_BM, F), lambda i: (i, 0)),
            pl.BlockSpec((_BM, 1), lambda i: (i, 0)),
            pl.BlockSpec((_BM, 1), lambda i: (i, 0)),
        ],
        out_shape=[
            jax.ShapeDtypeStruct((NP, F), jnp.float32),
            jax.ShapeDtypeStruct((NP, 1), jnp.float32),
            jax.ShapeDtypeStruct((NP, 1), jnp.float32),
        ],
    )(a0, a1, xw, d0, d1, b, wp)


def _topk_body(s0_ref, s1_ref, d0_ref, d1_ref, sp_ref, bp_ref, valid_ref,
               batch_ref, score_ref, sel_ref):
    dinv = lax.rsqrt(d0_ref[...] + d1_ref[...] + 1.0)
    score = dinv * (s0_ref[...] + s1_ref[...]) + dinv * dinv * sp_ref[...] \
        + bp_ref[...]
    score_ref[...] = score

    bits = lax.bitcast_convert_type(score, jnp.uint32)
    key = jnp.where(bits >> 31 != 0, ~bits, bits | jnp.uint32(0x80000000))

    gidx = lax.broadcasted_iota(jnp.int32, (G, 1), 0)
    gm = (batch_ref[...] == gidx) & (valid_ref[...] > 0)
    counts = jnp.sum(gm.astype(jnp.int32), axis=1, keepdims=True)
    k = (counts + 1) // 2

    def bit_step(t, prefix):
        b = (31 - t).astype(jnp.uint32)
        cand = prefix | lax.shift_left(jnp.uint32(1), b)
        cnt = jnp.sum((gm & (key >= cand)).astype(jnp.int32), axis=1,
                      keepdims=True)
        return jnp.where(cnt >= k, cand, prefix)

    T = lax.fori_loop(0, 32, bit_step, jnp.zeros((G, 1), jnp.uint32))
    cnt_gt = jnp.sum((gm & (key > T)).astype(jnp.int32), axis=1, keepdims=True)
    m = k - cnt_gt
    tie = gm & (key == T)
    idxv = lax.broadcasted_iota(jnp.int32, (1, NP), 1)

    def idx_step(t, ipref):
        cand = ipref | lax.shift_left(1, 13 - t)
        cnt_less = jnp.sum((tie & (idxv < cand)).astype(jnp.int32), axis=1,
                           keepdims=True)
        return jnp.where(cnt_less < m, cand, ipref)

    istar = lax.fori_loop(0, 14, idx_step, jnp.zeros((G, 1), jnp.int32))
    sel_g = (gm & (key > T)) | (tie & (idxv <= istar) & (m > 0))
    sel_ref[...] = jnp.any(sel_g, axis=0, keepdims=True).astype(jnp.float32)


def _tc_topk(s0, s1, d0, d1, sp, bp, valid, batch):
    return pl.pallas_call(
        _topk_body,
        out_shape=[
            jax.ShapeDtypeStruct((1, NP), jnp.float32),
            jax.ShapeDtypeStruct((1, NP), jnp.float32),
        ],
    )(s0, s1, d0, d1, sp, bp, valid, batch)


def _gate_mm_body(h_ref, s_ref, w_ref, hg_ref, xw2_ref):
    hg = h_ref[...] * jnp.tanh(s_ref[...])
    hg_ref[...] = hg
    xw2_ref[...] = jnp.dot(hg, w_ref[...], preferred_element_type=jnp.float32)


def _tc_gate_mm(h, score, W):
    return pl.pallas_call(
        _gate_mm_body,
        grid=(_GRID,),
        in_specs=[
            pl.BlockSpec((_BM, F), lambda i: (i, 0)),
            pl.BlockSpec((_BM, 1), lambda i: (i, 0)),
            pl.BlockSpec((F, F), lambda i: (0, 0)),
        ],
        out_specs=[
            pl.BlockSpec((_BM, F), lambda i: (i, 0)),
            pl.BlockSpec((_BM, F), lambda i: (i, 0)),
        ],
        out_shape=[
            jax.ShapeDtypeStruct((NP, F), jnp.float32),
            jax.ShapeDtypeStruct((NP, F), jnp.float32),
        ],
    )(h, score, W)


def _scale_body(xw_ref, d0_ref, d1_ref, y_ref):
    dinv = lax.rsqrt(d0_ref[...] + d1_ref[...] + 1.0)
    y_ref[...] = dinv * xw_ref[...]


def _tc_scale(xw, d0, d1):
    return pl.pallas_call(
        _scale_body,
        grid=(_GRID,),
        in_specs=[
            pl.BlockSpec((_BM, F), lambda i: (i, 0)),
            pl.BlockSpec((_BM, 1), lambda i: (i, 0)),
            pl.BlockSpec((_BM, 1), lambda i: (i, 0)),
        ],
        out_specs=pl.BlockSpec((_BM, F), lambda i: (i, 0)),
        out_shape=jax.ShapeDtypeStruct((NP, F), jnp.float32),
    )(xw, d0, d1)


def _gate_body(h_ref, s_ref, hg_ref):
    hg_ref[...] = h_ref[...] * jnp.tanh(s_ref[...])


def _tc_gate(h, score):
    return pl.pallas_call(
        _gate_body,
        grid=(_GRID,),
        in_specs=[
            pl.BlockSpec((_BM, F), lambda i: (i, 0)),
            pl.BlockSpec((_BM, 1), lambda i: (i, 0)),
        ],
        out_specs=pl.BlockSpec((_BM, F), lambda i: (i, 0)),
        out_shape=jax.ShapeDtypeStruct((NP, F), jnp.float32),
    )(h, score)


def _gmp_body_first(hg_ref, sel_ref, batch_ref, out_ref):
    _gmp_common(hg_ref, sel_ref, batch_ref, out_ref, None)


def _gmp_body_final(hg_ref, sel_ref, batch_ref, xprev_ref, out_ref):
    _gmp_common(hg_ref, sel_ref, batch_ref, out_ref, xprev_ref)


def _gmp_common(hg_ref, sel_ref, batch_ref, out_ref, xprev_ref):
    hg = hg_ref[...]
    selm = sel_ref[...] > 0
    rows = []
    for g in range(G):
        mask = (batch_ref[...] == g) & selm
        mx = jnp.max(jnp.where(mask, hg, -jnp.inf), axis=0, keepdims=True)
        sm = jnp.sum(jnp.where(mask, hg, 0.0), axis=0, keepdims=True)
        cnt = jnp.maximum(jnp.sum(mask.astype(jnp.float32)), 1.0)
        rows.append(jnp.concatenate([mx, sm / cnt], axis=1))
    res = jnp.concatenate(rows, axis=0)
    if xprev_ref is not None:
        res = (xprev_ref[...] + res) * 0.5
    out_ref[...] = res


def _tc_gmp_first(hg, sel, batch):
    return pl.pallas_call(
        _gmp_body_first,
        out_shape=jax.ShapeDtypeStruct((G, 2 * F), jnp.float32),
    )(hg, sel, batch)


def _tc_gmp_final(hg, sel, batch, xprev):
    return pl.pallas_call(
        _gmp_body_final,
        out_shape=jax.ShapeDtypeStruct((G, 2 * F), jnp.float32),
    )(hg, sel, batch, xprev)


def kernel(x, edge_index, edge_attr, batch, W1, b1, Wp1, bp1, W2, b2, Wp2, bp2):
    f32 = jnp.float32
    row = edge_index[0]
    col = edge_index[1]
    epad = EPAD - E
    rowp = jnp.concatenate([row, jnp.zeros((epad,), row.dtype)])
    colp = jnp.concatenate([col, jnp.zeros((epad,), col.dtype)])
    e2 = jnp.concatenate([rowp.reshape(EB, 1, 128), colp.reshape(EB, 1, 128)],
                         axis=1)
    wp_ = jnp.concatenate([edge_attr, jnp.zeros((epad,), f32)]).reshape(EB, 128)
    xp = jnp.concatenate([x, jnp.zeros((NP - N, F), f32)], axis=0)
    batchp = jnp.concatenate([batch, jnp.full((NP - N,), G, batch.dtype)])
    batch_r = batchp.reshape(1, NP)
    batch_c = batchp.reshape(NP, 1)
    z1d = jnp.zeros((RPT,), f32)
    z2d = jnp.zeros((RPT, F), f32)
    ones_r = jnp.ones((1, NP), f32)
    b1_2 = b1.reshape(1, F)
    b2_2 = b2.reshape(1, F)
    wp1_2 = Wp1.reshape(1, F)
    wp2_2 = Wp2.reshape(1, F)
    bp1_2 = bp1.reshape(1, 1)
    bp2_2 = bp2.reshape(1, 1)

    xw1 = _tc_mm(xp, W1)
    deg_p = _sc_deg(e2, wp_, z1d)
    d0 = deg_p[0].reshape(NP, 1)
    d1 = deg_p[1].reshape(NP, 1)
    y1 = _tc_scale(xw1, d0, d1)
    acc_p = _sc_feat(y1, e2, wp_, z2d)
    h, sp1, z1v = _tc_post(acc_p[0], acc_p[1], xw1, d0, d1, b1_2, wp1_2)
    sacc_p = _sc_smsg(z1v.reshape(NP), e2, wp_, z1d)
    score1, sel1 = _tc_topk(sacc_p[0].reshape(1, NP), sacc_p[1].reshape(1, NP),
                            d0.reshape(1, NP), d1.reshape(1, NP),
                            sp1.reshape(1, NP), bp1_2, ones_r, batch_r)
    hg, xw2 = _tc_gate_mm(h, score1.reshape(NP, 1), W2)
    x1 = _tc_gmp_first(hg, sel1.reshape(NP, 1), batch_c)

    w2, deg2_p = _sc_w2deg(sel1.reshape(NP), e2, wp_, z1d)
    e0 = deg2_p[0].reshape(NP, 1)
    e1 = deg2_p[1].reshape(NP, 1)
    y2 = _tc_scale(xw2, e0, e1)
    acc2_p = _sc_feat(y2, e2, w2, z2d)
    h2, sp2, z2v = _tc_post(acc2_p[0], acc2_p[1], xw2, e0, e1, b2_2, wp2_2)
    sacc2_p = _sc_smsg(z2v.reshape(NP), e2, w2, z1d)
    score2, sel2 = _tc_topk(sacc2_p[0].reshape(1, NP),
                            sacc2_p[1].reshape(1, NP),
                            e0.reshape(1, NP), e1.reshape(1, NP),
                            sp2.reshape(1, NP), bp2_2, sel1, batch_r)
    h3g = _tc_gate(h2, score2.reshape(NP, 1))
    return _tc_gmp_final(h3g, sel2.reshape(NP, 1), batch_c, x1)
